# baseline (speedup 1.0000x reference)
import functools

import jax
import jax.numpy as jnp
from jax import lax
from jax.experimental import pallas as pl
from jax.experimental.pallas import tpu as pltpu
from jax.experimental.pallas import tpu_sc as plsc

K = 8192
D = 32
N_BLOCK = 1024
K_BLOCK = 1024


def _argmax_body(x_ref, emb_ref, idx_ref, dots_ref):
    x = x_ref[...]
    acc_val = jnp.full((8, N_BLOCK), -jnp.inf, jnp.float32)
    acc_g = jnp.zeros((8, N_BLOCK), jnp.int32)
    for c in range(K // K_BLOCK):
        e = emb_ref[pl.ds(c * K_BLOCK, K_BLOCK), :]
        dots_ref[c % 2, :, :] = lax.dot_general(
            e, x, (((1,), (1,)), ((), ())),
            preferred_element_type=jnp.float32)

        def fold(g, carry, c=c):
            av, ag = carry
            v = dots_ref[c % 2, pl.ds(g * 8, 8), :]
            cmp = v > av
            return (jnp.maximum(av, v),
                    jnp.where(cmp, c * (K_BLOCK // 8) + g, ag))

        acc_val, acc_g = lax.fori_loop(0, K_BLOCK // 8, fold,
                                       (acc_val, acc_g), unroll=128)
    s_iota = lax.broadcasted_iota(jnp.int32, (8, N_BLOCK), 0)
    av, ai = acc_val, acc_g * 8 + s_iota
    for h in (4, 2, 1):
        v1, v2 = av[:h], av[h:2 * h]
        i1, i2 = ai[:h], ai[h:2 * h]
        pick1 = (v1 > v2) | ((v1 == v2) & (i1 < i2))
        av = jnp.where(pick1, v1, v2)
        ai = jnp.where(pick1, i1, i2)
    idx_ref[...] = ai.reshape(1, 1, N_BLOCK)


def _fused_argmax(flat, embedding):
    n = flat.shape[0]
    grid = n // N_BLOCK
    out = pl.pallas_call(
        _argmax_body,
        grid=(grid,),
        in_specs=[
            pl.BlockSpec((N_BLOCK, D), lambda i: (i, 0)),
            pl.BlockSpec((K, D), lambda i: (0, 0)),
        ],
        out_specs=pl.BlockSpec((1, 1, N_BLOCK), lambda i: (i, 0, 0)),
        out_shape=jax.ShapeDtypeStruct((grid, 1, N_BLOCK), jnp.int32),
        scratch_shapes=[pltpu.VMEM((2, K_BLOCK, N_BLOCK), jnp.float32)],
        compiler_params=pltpu.CompilerParams(
            dimension_semantics=("parallel",)),
    )(flat, embedding)
    return out.reshape(n)


def _make_sc_gather_bincount(n):
    info = plsc.get_sparse_core_info()
    nc, ns = info.num_cores, info.num_subcores
    nw = nc * ns
    b_per_w = n // nw
    n_chunks = b_per_w // 128

    mesh = plsc.VectorSubcoreMesh(core_axis_name="c", subcore_axis_name="s")

    @functools.partial(
        pl.kernel, mesh=mesh,
        compiler_params=pltpu.CompilerParams(use_tc_tiling_on_sc=False),
        out_type=[
            jax.ShapeDtypeStruct((n, D), jnp.float32),
            jax.ShapeDtypeStruct((nc, K), jnp.float32),
        ],
        scratch_types=[
            pltpu.VMEM((n_chunks, 128), jnp.int32),
            pltpu.VMEM((b_per_w, D), jnp.float32),
            pltpu.VMEM((128,), jnp.float32),
            pltpu.VMEM_SHARED((K,), jnp.float32),
            pltpu.SemaphoreType.DMA,
        ],
    )
    def sc_kernel(emb_hbm, idx_hbm, zeros_hbm, zq_hbm, usage_hbm,
                  idx_v, rows_v, ones_v, usage_sh, sem):
        cid = lax.axis_index("c")
        sid = lax.axis_index("s")
        wid = sid * nc + cid
        pltpu.sync_copy(idx_hbm.at[pl.ds(wid * n_chunks, n_chunks)], idx_v)

        @pl.when(sid == 0)
        def _():
            pltpu.sync_copy(zeros_hbm, usage_sh)

        for t in range(8):
            ones_v[pl.ds(t * 16, 16)] = jnp.full((16,), 1.0, jnp.float32)

        copies = [
            pltpu.async_copy(emb_hbm.at[idx_v.at[j]],
                             rows_v.at[pl.ds(j * 128, 128)], sem)
            for j in range(n_chunks)
        ]
        plsc.subcore_barrier()
        for j in range(n_chunks):
            pltpu.sync_copy(ones_v, usage_sh.at[idx_v.at[j]], add=True)
        for cp in copies:
            cp.wait()
        pltpu.sync_copy(rows_v, zq_hbm.at[pl.ds(wid * b_per_w, b_per_w)])
        plsc.subcore_barrier()

        @pl.when(sid == 0)
        def _():
            pltpu.sync_copy(usage_sh, usage_hbm.at[cid])

    return sc_kernel


def _stats_body(u_ref, out_ref):
    u = jnp.sum(u_ref[...], axis=0, keepdims=True)
    total = jnp.sum(u)
    probs = u / jnp.maximum(total, 1.0)
    safe = jnp.where(probs > 0, probs, 1.0)
    perp = jnp.exp(-jnp.sum(probs * jnp.log(safe)))
    dead = jnp.mean((u == 0).astype(jnp.float32))
    out_ref[...] = jnp.concatenate(
        [perp.reshape(1, 1), dead.reshape(1, 1)], axis=1)


def kernel(z_e, embedding):
    Bv, Lv, Dv = z_e.shape
    n = Bv * Lv
    flat = z_e.reshape(-1, Dv)
    norm = jnp.clip(jnp.linalg.norm(flat, axis=1, keepdims=True), 1e-08)
    flat_norm = (flat / norm).astype(jnp.bfloat16)
    indices = _fused_argmax(flat_norm, embedding.astype(jnp.bfloat16))

    sc = _make_sc_gather_bincount(n)
    zq_flat, usage_parts = sc(embedding, indices.reshape(-1, 128),
                              jnp.zeros((K,), jnp.float32))
    z_q = zq_flat.reshape(Bv, Lv, Dv)

    stats = pl.pallas_call(
        _stats_body,
        out_shape=jax.ShapeDtypeStruct((1, 2), jnp.float32),
    )(usage_parts).reshape(2)
    return (z_q, z_q, indices.reshape(Bv, Lv), stats)


# speedup vs baseline: 1.0146x; 1.0146x over previous
import functools

import jax
import jax.numpy as jnp
from jax import lax
from jax.experimental import pallas as pl
from jax.experimental.pallas import tpu as pltpu
from jax.experimental.pallas import tpu_sc as plsc

K = 8192
D = 32
N_BLOCK = 2048
K_BLOCK = 512


def _argmax_body(x_ref, emb_ref, idx_ref, dots_ref):
    x = x_ref[...]
    acc_val = jnp.full((8, N_BLOCK), -jnp.inf, jnp.float32)
    acc_g = jnp.zeros((8, N_BLOCK), jnp.int32)
    for c in range(K // K_BLOCK):
        e = emb_ref[pl.ds(c * K_BLOCK, K_BLOCK), :]
        dots_ref[c % 2, :, :] = lax.dot_general(
            e, x, (((1,), (1,)), ((), ())),
            preferred_element_type=jnp.float32)

        def fold(g, carry, c=c):
            av, ag = carry
            v = dots_ref[c % 2, pl.ds(g * 8, 8), :]
            cmp = v > av
            return (jnp.maximum(av, v),
                    jnp.where(cmp, c * (K_BLOCK // 8) + g, ag))

        acc_val, acc_g = lax.fori_loop(0, K_BLOCK // 8, fold,
                                       (acc_val, acc_g), unroll=128)
    s_iota = lax.broadcasted_iota(jnp.int32, (8, N_BLOCK), 0)
    av, ai = acc_val, acc_g * 8 + s_iota
    for h in (4, 2, 1):
        v1, v2 = av[:h], av[h:2 * h]
        i1, i2 = ai[:h], ai[h:2 * h]
        pick1 = (v1 > v2) | ((v1 == v2) & (i1 < i2))
        av = jnp.where(pick1, v1, v2)
        ai = jnp.where(pick1, i1, i2)
    idx_ref[...] = ai.reshape(1, 1, N_BLOCK)


def _fused_argmax(flat, embedding):
    n = flat.shape[0]
    grid = n // N_BLOCK
    out = pl.pallas_call(
        _argmax_body,
        grid=(grid,),
        in_specs=[
            pl.BlockSpec((N_BLOCK, D), lambda i: (i, 0)),
            pl.BlockSpec((K, D), lambda i: (0, 0)),
        ],
        out_specs=pl.BlockSpec((1, 1, N_BLOCK), lambda i: (i, 0, 0)),
        out_shape=jax.ShapeDtypeStruct((grid, 1, N_BLOCK), jnp.int32),
        scratch_shapes=[pltpu.VMEM((2, K_BLOCK, N_BLOCK), jnp.float32)],
        compiler_params=pltpu.CompilerParams(
            dimension_semantics=("parallel",)),
    )(flat, embedding)
    return out.reshape(n)


def _make_sc_gather_bincount(n):
    info = plsc.get_sparse_core_info()
    nc, ns = info.num_cores, info.num_subcores
    nw = nc * ns
    b_per_w = n // nw
    n_chunks = b_per_w // 128

    mesh = plsc.VectorSubcoreMesh(core_axis_name="c", subcore_axis_name="s")

    @functools.partial(
        pl.kernel, mesh=mesh,
        compiler_params=pltpu.CompilerParams(use_tc_tiling_on_sc=False),
        out_type=[
            jax.ShapeDtypeStruct((n, D), jnp.float32),
            jax.ShapeDtypeStruct((nc, K), jnp.float32),
        ],
        scratch_types=[
            pltpu.VMEM((n_chunks, 128), jnp.int32),
            pltpu.VMEM((b_per_w, D), jnp.float32),
            pltpu.VMEM((128,), jnp.float32),
            pltpu.VMEM_SHARED((K,), jnp.float32),
            pltpu.SemaphoreType.DMA,
        ],
    )
    def sc_kernel(emb_hbm, idx_hbm, zeros_hbm, zq_hbm, usage_hbm,
                  idx_v, rows_v, ones_v, usage_sh, sem):
        cid = lax.axis_index("c")
        sid = lax.axis_index("s")
        wid = sid * nc + cid
        pltpu.sync_copy(idx_hbm.at[pl.ds(wid * n_chunks, n_chunks)], idx_v)

        @pl.when(sid == 0)
        def _():
            pltpu.sync_copy(zeros_hbm, usage_sh)

        for t in range(8):
            ones_v[pl.ds(t * 16, 16)] = jnp.full((16,), 1.0, jnp.float32)

        copies = [
            pltpu.async_copy(emb_hbm.at[idx_v.at[j]],
                             rows_v.at[pl.ds(j * 128, 128)], sem)
            for j in range(n_chunks)
        ]
        plsc.subcore_barrier()
        for j in range(n_chunks):
            pltpu.sync_copy(ones_v, usage_sh.at[idx_v.at[j]], add=True)
        for cp in copies:
            cp.wait()
        pltpu.sync_copy(rows_v, zq_hbm.at[pl.ds(wid * b_per_w, b_per_w)])
        plsc.subcore_barrier()

        @pl.when(sid == 0)
        def _():
            pltpu.sync_copy(usage_sh, usage_hbm.at[cid])

    return sc_kernel


def _stats_body(u_ref, out_ref):
    u = jnp.sum(u_ref[...], axis=0, keepdims=True)
    total = jnp.sum(u)
    probs = u / jnp.maximum(total, 1.0)
    safe = jnp.where(probs > 0, probs, 1.0)
    perp = jnp.exp(-jnp.sum(probs * jnp.log(safe)))
    dead = jnp.mean((u == 0).astype(jnp.float32))
    out_ref[...] = jnp.concatenate(
        [perp.reshape(1, 1), dead.reshape(1, 1)], axis=1)


def kernel(z_e, embedding):
    Bv, Lv, Dv = z_e.shape
    n = Bv * Lv
    flat = z_e.reshape(-1, Dv)
    norm = jnp.clip(jnp.linalg.norm(flat, axis=1, keepdims=True), 1e-08)
    flat_norm = (flat / norm).astype(jnp.bfloat16)
    indices = _fused_argmax(flat_norm, embedding.astype(jnp.bfloat16))

    sc = _make_sc_gather_bincount(n)
    zq_flat, usage_parts = sc(embedding, indices.reshape(-1, 128),
                              jnp.zeros((K,), jnp.float32))
    z_q = zq_flat.reshape(Bv, Lv, Dv)

    stats = pl.pallas_call(
        _stats_body,
        out_shape=jax.ShapeDtypeStruct((1, 2), jnp.float32),
    )(usage_parts).reshape(2)
    return (z_q, z_q, indices.reshape(Bv, Lv), stats)


# speedup vs baseline: 1.0157x; 1.0011x over previous
import functools

import jax
import jax.numpy as jnp
from jax import lax
from jax.experimental import pallas as pl
from jax.experimental.pallas import tpu as pltpu
from jax.experimental.pallas import tpu_sc as plsc

K = 8192
D = 32
N_BLOCK = 2048
K_BLOCK = 1024


def _argmax_body(x_ref, emb_ref, idx_ref, dots_ref):
    x = x_ref[...]
    acc_val = jnp.full((8, N_BLOCK), -jnp.inf, jnp.float32)
    acc_g = jnp.zeros((8, N_BLOCK), jnp.int32)
    for c in range(K // K_BLOCK):
        e = emb_ref[pl.ds(c * K_BLOCK, K_BLOCK), :]
        dots_ref[c % 2, :, :] = lax.dot_general(
            e, x, (((1,), (1,)), ((), ())),
            preferred_element_type=jnp.float32)

        def fold(g, carry, c=c):
            av, ag = carry
            v = dots_ref[c % 2, pl.ds(g * 8, 8), :]
            cmp = v > av
            return (jnp.maximum(av, v),
                    jnp.where(cmp, c * (K_BLOCK // 8) + g, ag))

        acc_val, acc_g = lax.fori_loop(0, K_BLOCK // 8, fold,
                                       (acc_val, acc_g), unroll=128)
    s_iota = lax.broadcasted_iota(jnp.int32, (8, N_BLOCK), 0)
    av, ai = acc_val, acc_g * 8 + s_iota
    for h in (4, 2, 1):
        v1, v2 = av[:h], av[h:2 * h]
        i1, i2 = ai[:h], ai[h:2 * h]
        pick1 = (v1 > v2) | ((v1 == v2) & (i1 < i2))
        av = jnp.where(pick1, v1, v2)
        ai = jnp.where(pick1, i1, i2)
    idx_ref[...] = ai.reshape(1, 1, N_BLOCK)


def _fused_argmax(flat, embedding):
    n = flat.shape[0]
    grid = n // N_BLOCK
    out = pl.pallas_call(
        _argmax_body,
        grid=(grid,),
        in_specs=[
            pl.BlockSpec((N_BLOCK, D), lambda i: (i, 0)),
            pl.BlockSpec((K, D), lambda i: (0, 0)),
        ],
        out_specs=pl.BlockSpec((1, 1, N_BLOCK), lambda i: (i, 0, 0)),
        out_shape=jax.ShapeDtypeStruct((grid, 1, N_BLOCK), jnp.int32),
        scratch_shapes=[pltpu.VMEM((2, K_BLOCK, N_BLOCK), jnp.float32)],
        compiler_params=pltpu.CompilerParams(
            dimension_semantics=("parallel",)),
    )(flat, embedding)
    return out.reshape(n)


def _make_sc_gather_bincount(n):
    info = plsc.get_sparse_core_info()
    nc, ns = info.num_cores, info.num_subcores
    nw = nc * ns
    b_per_w = n // nw
    n_chunks = b_per_w // 128

    mesh = plsc.VectorSubcoreMesh(core_axis_name="c", subcore_axis_name="s")

    @functools.partial(
        pl.kernel, mesh=mesh,
        compiler_params=pltpu.CompilerParams(use_tc_tiling_on_sc=False),
        out_type=[
            jax.ShapeDtypeStruct((n, D), jnp.float32),
            jax.ShapeDtypeStruct((nc, K), jnp.float32),
        ],
        scratch_types=[
            pltpu.VMEM((n_chunks, 128), jnp.int32),
            pltpu.VMEM((b_per_w, D), jnp.float32),
            pltpu.VMEM((128,), jnp.float32),
            pltpu.VMEM_SHARED((K,), jnp.float32),
            pltpu.SemaphoreType.DMA,
        ],
    )
    def sc_kernel(emb_hbm, idx_hbm, zeros_hbm, zq_hbm, usage_hbm,
                  idx_v, rows_v, ones_v, usage_sh, sem):
        cid = lax.axis_index("c")
        sid = lax.axis_index("s")
        wid = sid * nc + cid
        pltpu.sync_copy(idx_hbm.at[pl.ds(wid * n_chunks, n_chunks)], idx_v)

        @pl.when(sid == 0)
        def _():
            pltpu.sync_copy(zeros_hbm, usage_sh)

        for t in range(8):
            ones_v[pl.ds(t * 16, 16)] = jnp.full((16,), 1.0, jnp.float32)

        copies = [
            pltpu.async_copy(emb_hbm.at[idx_v.at[j]],
                             rows_v.at[pl.ds(j * 128, 128)], sem)
            for j in range(n_chunks)
        ]
        plsc.subcore_barrier()
        for j in range(n_chunks):
            pltpu.sync_copy(ones_v, usage_sh.at[idx_v.at[j]], add=True)
        for cp in copies:
            cp.wait()
        pltpu.sync_copy(rows_v, zq_hbm.at[pl.ds(wid * b_per_w, b_per_w)])
        plsc.subcore_barrier()

        @pl.when(sid == 0)
        def _():
            pltpu.sync_copy(usage_sh, usage_hbm.at[cid])

    return sc_kernel


def _stats_body(u_ref, out_ref):
    u = jnp.sum(u_ref[...], axis=0, keepdims=True)
    total = jnp.sum(u)
    probs = u / jnp.maximum(total, 1.0)
    safe = jnp.where(probs > 0, probs, 1.0)
    perp = jnp.exp(-jnp.sum(probs * jnp.log(safe)))
    dead = jnp.mean((u == 0).astype(jnp.float32))
    out_ref[...] = jnp.concatenate(
        [perp.reshape(1, 1), dead.reshape(1, 1)], axis=1)


def kernel(z_e, embedding):
    Bv, Lv, Dv = z_e.shape
    n = Bv * Lv
    flat = z_e.reshape(-1, Dv)
    norm = jnp.clip(jnp.linalg.norm(flat, axis=1, keepdims=True), 1e-08)
    flat_norm = (flat / norm).astype(jnp.bfloat16)
    indices = _fused_argmax(flat_norm, embedding.astype(jnp.bfloat16))

    sc = _make_sc_gather_bincount(n)
    zq_flat, usage_parts = sc(embedding, indices.reshape(-1, 128),
                              jnp.zeros((K,), jnp.float32))
    z_q = zq_flat.reshape(Bv, Lv, Dv)

    stats = pl.pallas_call(
        _stats_body,
        out_shape=jax.ShapeDtypeStruct((1, 2), jnp.float32),
    )(usage_parts).reshape(2)
    return (z_q, z_q, indices.reshape(Bv, Lv), stats)
